# baseline (speedup 1.0000x reference)
import functools

import jax
import jax.numpy as jnp
from jax import lax
from jax.experimental import pallas as pl
from jax.experimental.pallas import tpu as pltpu
from jax.experimental.pallas import tpu_sc as plsc

NUM_EMBEDDINGS = 1000000
EMBEDDING_DIM = 64
N_INDICES = 16384

_NC = 2
_NS = 16
_NW = _NC * _NS
_B_PER_W = N_INDICES // _NW
_THETA = 624992
_SENTINEL = 2147483647

_mesh = plsc.VectorSubcoreMesh(core_axis_name="c", subcore_axis_name="s")
_out_type = jax.ShapeDtypeStruct((N_INDICES, EMBEDDING_DIM), jnp.float32)


@functools.partial(
    pl.kernel,
    mesh=_mesh,
    out_type=_out_type,
    scratch_types=[
        pltpu.VMEM((_B_PER_W,), jnp.int32),
        pltpu.VMEM((_B_PER_W, EMBEDDING_DIM), jnp.float32),
        pltpu.SemaphoreType.DMA,
    ],
)
def _tail_kernel(indice_hbm, table_hbm, out_hbm, idx_v, rows_v, sem):
    wid = lax.axis_index("s") * _NC + lax.axis_index("c")
    base = wid * _B_PER_W
    pltpu.sync_copy(indice_hbm.at[pl.ds(base, _B_PER_W)], idx_v)

    def fire(j, _):
        v = idx_v[pl.ds(j * 16, 16)]
        for k in range(16):
            r = v[k]

            @pl.when(r >= _THETA)
            def _():
                pltpu.make_async_copy(
                    table_hbm.at[r], rows_v.at[j * 16 + k], sem
                ).start()

        return 0

    lax.fori_loop(0, _B_PER_W // 16, fire, 0)

    def drain(j, _):
        v = idx_v[pl.ds(j * 16, 16)]
        for k in range(16):
            r = v[k]

            @pl.when(r >= _THETA)
            def _():
                pltpu.make_async_copy(
                    table_hbm.at[0], rows_v.at[0], sem
                ).wait()

        return 0

    lax.fori_loop(0, _B_PER_W // 16, drain, 0)
    pltpu.sync_copy(rows_v, out_hbm.at[pl.ds(base, _B_PER_W)])


@functools.partial(
    pl.kernel,
    mesh=_mesh,
    out_type=_out_type,
    scratch_types=[
        pltpu.VMEM((_B_PER_W,), jnp.int32),
        pltpu.VMEM((_B_PER_W, EMBEDDING_DIM), jnp.float32),
        pltpu.SemaphoreType.DMA,
    ],
    compiler_params=pltpu.CompilerParams(use_tc_tiling_on_sc=False),
)
def _head_kernel(indice_hbm, head_hbm, out_hbm, idx_v, rows_v, sem):
    wid = lax.axis_index("s") * _NC + lax.axis_index("c")
    base = wid * _B_PER_W
    pltpu.sync_copy(indice_hbm.at[pl.ds(base, _B_PER_W)], idx_v)

    def mask(j, _):
        v = idx_v[pl.ds(j * 16, 16)]
        v = lax.select(v < _THETA, v, lax.full_like(v, _SENTINEL))
        idx_v[pl.ds(j * 16, 16)] = v
        return 0

    lax.fori_loop(0, _B_PER_W // 16, mask, 0)
    pltpu.async_copy(
        head_hbm.at[plsc.Indices(idx_v, ignored_value=_SENTINEL)], rows_v, sem
    ).wait()
    pltpu.sync_copy(rows_v, out_hbm.at[pl.ds(base, _B_PER_W)])


def kernel(indice, table):
    idx32 = indice.astype(jnp.int32)
    head = table[:_THETA]
    tail_out = _tail_kernel(idx32, table)
    head_out = _head_kernel(idx32, head)
    return jnp.where((idx32 < _THETA)[:, None], head_out, tail_out)


# speedup vs baseline: 1.6452x; 1.6452x over previous
import functools

import jax
import jax.numpy as jnp
from jax import lax
from jax.experimental import pallas as pl
from jax.experimental.pallas import tpu as pltpu
from jax.experimental.pallas import tpu_sc as plsc

NUM_EMBEDDINGS = 1000000
EMBEDDING_DIM = 64
N_INDICES = 16384

_NC = 2
_NS = 16
_NW = _NC * _NS
_B_PER_W = N_INDICES // _NW
_THETA = 500000

_mesh = plsc.VectorSubcoreMesh(core_axis_name="c", subcore_axis_name="s")
_out_type = jax.ShapeDtypeStruct((N_INDICES, EMBEDDING_DIM), jnp.float32)


@functools.partial(
    pl.kernel,
    mesh=_mesh,
    out_type=_out_type,
    scratch_types=[
        pltpu.VMEM((_B_PER_W,), jnp.int32),
        pltpu.VMEM((_B_PER_W, EMBEDDING_DIM), jnp.float32),
        pltpu.SemaphoreType.DMA,
    ],
)
def _sc_tail_kernel(indice_hbm, table_hbm, out_hbm, idx_v, rows_v, sem):
    wid = lax.axis_index("s") * _NC + lax.axis_index("c")
    base = wid * _B_PER_W
    pltpu.sync_copy(indice_hbm.at[pl.ds(base, _B_PER_W)], idx_v)

    def fire(j, _):
        v = idx_v[pl.ds(j * 16, 16)]
        for k in range(16):
            r = v[k]

            @pl.when(r >= _THETA)
            def _():
                pltpu.make_async_copy(
                    table_hbm.at[r], rows_v.at[j * 16 + k], sem
                ).start()

        return 0

    lax.fori_loop(0, _B_PER_W // 16, fire, 0)

    def drain(j, _):
        v = idx_v[pl.ds(j * 16, 16)]
        for k in range(16):
            r = v[k]

            @pl.when(r >= _THETA)
            def _():
                pltpu.make_async_copy(
                    table_hbm.at[0], rows_v.at[0], sem
                ).wait()

        return 0

    lax.fori_loop(0, _B_PER_W // 16, drain, 0)
    pltpu.sync_copy(rows_v, out_hbm.at[pl.ds(base, _B_PER_W)])


_TC_GRID = 16
_TC_B = N_INDICES // _TC_GRID


def _tc_head_body(idx_smem, table_hbm, out_vmem, sem):
    def fire(i, _):
        r = idx_smem[i]

        @pl.when(r < _THETA)
        def _():
            pltpu.make_async_copy(
                table_hbm.at[r], out_vmem.at[i], sem
            ).start()

        return 0

    lax.fori_loop(0, _TC_B, fire, 0, unroll=8)

    def drain(i, _):
        r = idx_smem[i]

        @pl.when(r < _THETA)
        def _():
            pltpu.make_async_copy(
                table_hbm.at[0], out_vmem.at[0], sem
            ).wait()

        return 0

    lax.fori_loop(0, _TC_B, drain, 0, unroll=8)


_tc_head_kernel = pl.pallas_call(
    _tc_head_body,
    grid=(_TC_GRID,),
    in_specs=[
        pl.BlockSpec((_TC_B,), lambda g: (g,), memory_space=pltpu.SMEM),
        pl.BlockSpec(memory_space=pl.ANY),
    ],
    out_specs=pl.BlockSpec((_TC_B, EMBEDDING_DIM), lambda g: (g, 0)),
    out_shape=_out_type,
    scratch_shapes=[pltpu.SemaphoreType.DMA],
)


def kernel(indice, table):
    idx32 = indice.astype(jnp.int32)
    tail_out = _sc_tail_kernel(idx32, table)
    head_out = _tc_head_kernel(idx32, table)
    return jnp.where((idx32 < _THETA)[:, None], head_out, tail_out)


# speedup vs baseline: 2.1897x; 1.3310x over previous
import functools

import jax
import jax.numpy as jnp
from jax import lax
from jax.experimental import pallas as pl
from jax.experimental.pallas import tpu as pltpu
from jax.experimental.pallas import tpu_sc as plsc

NUM_EMBEDDINGS = 1000000
EMBEDDING_DIM = 64
N_INDICES = 16384

_NC = 2
_NS = 16
_NW = _NC * _NS
_B_PER_W = N_INDICES // _NW

_mesh = plsc.VectorSubcoreMesh(core_axis_name="c", subcore_axis_name="s")


@functools.partial(
    pl.kernel,
    mesh=_mesh,
    out_type=jax.ShapeDtypeStruct((N_INDICES, EMBEDDING_DIM), jnp.float32),
    scratch_types=[
        pltpu.VMEM((_B_PER_W,), jnp.int32),
        pltpu.VMEM((_B_PER_W, EMBEDDING_DIM), jnp.float32),
        pltpu.SemaphoreType.DMA,
        pltpu.SemaphoreType.DMA,
        pltpu.SemaphoreType.DMA,
        pltpu.SemaphoreType.DMA,
    ],
)
def _gather_kernel(indice_hbm, table_hbm, out_hbm, idx_v, rows_v,
                   sem0, sem1, sem2, sem3):
    sems = (sem0, sem1, sem2, sem3)
    wid = lax.axis_index("s") * _NC + lax.axis_index("c")
    base = wid * _B_PER_W
    pltpu.sync_copy(indice_hbm.at[pl.ds(base, _B_PER_W)], idx_v)

    def fire(j, _):
        v = idx_v[pl.ds(j * 16, 16)]
        for k in range(16):
            r = v[k]
            pltpu.make_async_copy(
                table_hbm.at[r], rows_v.at[j * 16 + k], sems[k % 4]
            ).start()
        return 0

    lax.fori_loop(0, _B_PER_W // 16, fire, 0)

    def drain(j, _):
        for k in range(16):
            pltpu.make_async_copy(
                table_hbm.at[0], rows_v.at[0], sems[k % 4]
            ).wait()
        return 0

    lax.fori_loop(0, _B_PER_W // 16, drain, 0)
    pltpu.sync_copy(rows_v, out_hbm.at[pl.ds(base, _B_PER_W)])


def kernel(indice, table):
    return _gather_kernel(indice.astype(jnp.int32), table)
